# baseline (speedup 1.0000x reference)
import functools

import jax
import jax.numpy as jnp
import numpy as np
from jax import lax
from jax.experimental import pallas as pl
from jax.experimental.pallas import tpu as pltpu
from jax.experimental.pallas import tpu_sc as plsc

_B, _T, _D, _M = 16, 128, 64, 8192
_N = _B * _T
_DECAY = 0.99
_EPS = 1e-05
_LOG_M = float(np.log(_M))


def _stage_a_body(a_ref, v_ref, e_ref, x2_ref, e2_ref,
                  idx_ref, ph_ref, cnt_ref, dw_ref, bs_ref):
    i = pl.program_id(0)
    mod = i // _B
    b = i % _B

    @pl.when(i == 0)
    def _init():
        cnt_ref[...] = jnp.zeros_like(cnt_ref)
        dw_ref[...] = jnp.zeros_like(dw_ref)

    a = a_ref[b]
    v = v_ref[b]
    x = jnp.where(mod == 0, a, v)
    sumflat = a + v
    e = e_ref[...]

    e2 = e2_ref[0]
    x2 = x2_ref[mod, b]
    xe = lax.dot_general(x, e, (((1,), (1,)), ((), ())),
                         preferred_element_type=jnp.float32)
    d = (e2[None, :] + x2[:, None]) - 2.0 * xe

    miota = lax.broadcasted_iota(jnp.int32, (_T, _M), 1)
    dmin = jnp.min(d, axis=1, keepdims=True)
    idx = jnp.min(jnp.where(d == dmin, miota, _M), axis=1).astype(jnp.int32)

    s = -jnp.sqrt(jnp.maximum(d, 0.0))
    smax = -jnp.sqrt(jnp.maximum(dmin, 0.0))
    u = jnp.exp(s - smax)
    z = jnp.sum(u, axis=1, keepdims=True)
    p = u / z
    ent = -jnp.sum(p * jnp.log(p + 1e-5), axis=1)
    adj = 1.0 - ent / _LOG_M
    ph_b = jnp.mean(p, axis=0)
    cons_b = jnp.sum(jnp.abs(p - ph_b[None, :]))

    one = (miota == idx[:, None]).astype(jnp.float32)
    y = adj[:, None] * sumflat
    ones_col = jnp.ones((_T, 1), jnp.float32)
    y2 = jnp.concatenate([y, ones_col, adj[:, None]], axis=1)
    dwc2 = lax.dot_general(one, y2, (((0,), (0,)), ((), ())),
                           preferred_element_type=jnp.float32)
    dwc = dwc2[:, :_D]
    cnt_vec = dwc2[:, _D]
    wcnt = dwc2[:, _D + 1]
    viota = lax.broadcasted_iota(jnp.int32, (1, _M), 1).reshape(_M)
    cmax = jnp.max(cnt_vec)
    mode = jnp.min(jnp.where(cnt_vec == cmax, viota, _M)).astype(jnp.float32)

    idx_ref[mod, b] = idx
    ph_ref[mod, b] = ph_b
    cnt_ref[mod] = cnt_ref[mod] + wcnt
    dw_ref[mod] = dw_ref[mod] + dwc
    li = lax.broadcasted_iota(jnp.int32, (1, _T), 1).reshape(_T)
    bs_ref[mod, b] = jnp.where(li == 0, cons_b,
                               jnp.where(li == 1, mode, 0.0))


def _stage_a(audio, video, embedding, x2, e2):
    full = lambda shape: pl.BlockSpec(shape, lambda i: (0,) * len(shape))
    return pl.pallas_call(
        _stage_a_body,
        grid=(2 * _B,),
        in_specs=[full((_B, _T, _D)), full((_B, _T, _D)), full((_M, _D)),
                  full((2, _B, _T)), full((1, _M))],
        out_specs=[full((2, _B, _T)), full((2, _B, _M)), full((2, _M)),
                   full((2, _M, _D)), full((2, _B, _T))],
        out_shape=[
            jax.ShapeDtypeStruct((2, _B, _T), jnp.int32),
            jax.ShapeDtypeStruct((2, _B, _M), jnp.float32),
            jax.ShapeDtypeStruct((2, _M), jnp.float32),
            jax.ShapeDtypeStruct((2, _M, _D), jnp.float32),
            jax.ShapeDtypeStruct((2, _B, _T), jnp.float32),
        ],
        compiler_params=pltpu.CompilerParams(
            dimension_semantics=("arbitrary",)),
    )(audio, video, embedding, x2, e2)


_NW = 32
_ROWS_PER_W = 2 * _N // _NW
_DP = 128


def _sc_gather(table_padded, idx_flat):
    mesh = plsc.VectorSubcoreMesh(core_axis_name="c", subcore_axis_name="s")

    @functools.partial(
        pl.kernel, mesh=mesh,
        out_type=jax.ShapeDtypeStruct((2 * _N, _DP), jnp.float32),
        scratch_types=[
            pltpu.VMEM((_ROWS_PER_W,), jnp.int32),
            pltpu.VMEM((_ROWS_PER_W, _DP), jnp.float32),
            pltpu.SemaphoreType.DMA,
        ],
    )
    def gather_k(table_hbm, idx_hbm, out_hbm, idx_v, rows_v, sem):
        wid = lax.axis_index("s") * 2 + lax.axis_index("c")
        base = wid * _ROWS_PER_W
        pltpu.sync_copy(idx_hbm.at[pl.ds(base, _ROWS_PER_W)], idx_v)
        pltpu.async_copy(table_hbm.at[idx_v], rows_v, sem).wait()
        pltpu.sync_copy(rows_v, out_hbm.at[pl.ds(base, _ROWS_PER_W)])

    return gather_k(table_padded, idx_flat)


def _stage_c_body(cnt_ref, dw_ref, ph_ref, bs_ref, ec_ref, ew_ref,
                  a_ref, v_ref, aq_ref, vq_ref,
                  emb_ref, sc_ref):
    ec = ec_ref[0]
    a_n = cnt_ref[0]
    v_n = cnt_ref[1]

    ec1 = _DECAY * ec + (1.0 - _DECAY) * v_n
    n1 = jnp.sum(ec1)
    ec1 = (ec1 + _EPS) / (n1 + _M * _EPS) * n1
    ew1 = _DECAY * ew_ref[...] + 0.5 * (1.0 - _DECAY) * dw_ref[1]
    ec2 = _DECAY * ec1 + (1.0 - _DECAY) * a_n
    n2 = jnp.sum(ec2)
    ec2 = (ec2 + _EPS) / (n2 + _M * _EPS) * n2
    ew2 = _DECAY * ew1 + 0.5 * (1.0 - _DECAY) * dw_ref[0]
    emb_ref[...] = ew2 / ec2[:, None]

    a_ph = ph_ref[0]
    v_ph = ph_ref[1]
    la = jnp.log(a_ph + 1e-10)
    lv = jnp.log(v_ph + 1e-10)
    scode = (lax.dot_general(a_ph, lv, (((1,), (1,)), ((), ())),
                             preferred_element_type=jnp.float32)
             + lax.dot_general(v_ph, la, (((1,), (1,)), ((), ())),
                               preferred_element_type=jnp.float32))
    mx = jnp.max(-scode)
    es = jnp.exp(scode + mx)
    rs = jnp.sum(es, axis=1)
    r = lax.broadcasted_iota(jnp.int32, (_B, _B), 0)
    c = lax.broadcasted_iota(jnp.int32, (_B, _B), 1)
    diag = jnp.sum(jnp.where(r == c, es, 0.0), axis=1)
    lcmcm = -jnp.sum(jnp.log(diag / (rs + _EPS))) / _B

    bs = bs_ref[...]
    mi = lax.broadcasted_iota(jnp.int32, (2, _B, _T), 0)
    li = lax.broadcasted_iota(jnp.int32, (2, _B, _T), 2)
    a_cons = jnp.sum(jnp.where((li == 0) & (mi == 0), bs, 0.0)) / _B
    v_cons = jnp.sum(jnp.where((li == 0) & (mi == 1), bs, 0.0)) / _B
    modes = jnp.sum(jnp.where(li == 1, bs, 0.0), axis=2)
    eqn = jnp.sum(jnp.where(modes[0] == modes[1], 1.0, 0.0))

    a_loss = 0.5 * jnp.mean((a_ref[...] - aq_ref[...]) ** 2)
    v_loss = 0.5 * jnp.mean((v_ref[...] - vq_ref[...]) ** 2)

    lo = lax.broadcasted_iota(jnp.int32, (1, 128), 1)
    sc_ref[...] = jnp.where(lo == 0, a_loss,
                  jnp.where(lo == 1, v_loss,
                  jnp.where(lo == 2, lcmcm,
                  jnp.where(lo == 3, a_cons,
                  jnp.where(lo == 4, v_cons,
                  jnp.where(lo == 5, eqn, 0.0))))))


def _stage_c(cnt, dw, ph, bs, ema_count, ema_weight, audio, video, aq, vq):
    full = lambda shape: pl.BlockSpec(shape, lambda: (0,) * len(shape))
    return pl.pallas_call(
        _stage_c_body,
        in_specs=[full((2, _M)), full((2, _M, _D)), full((2, _B, _M)),
                  full((2, _B, _T)), full((1, _M)), full((_M, _D)),
                  full((_B, _T, _D)), full((_B, _T, _D)),
                  full((_B, _T, _D)), full((_B, _T, _D))],
        out_specs=[full((_M, _D)), full((1, 128))],
        out_shape=[
            jax.ShapeDtypeStruct((_M, _D), jnp.float32),
            jax.ShapeDtypeStruct((1, 128), jnp.float32),
        ],
    )(cnt, dw, ph, bs, ema_count, ema_weight, audio, video, aq, vq)


def kernel(audio_semantic, video_semantic, epoch, embedding, ema_count,
           ema_weight, coefficients):
    audio = audio_semantic.astype(jnp.float32)
    video = video_semantic.astype(jnp.float32)
    emb = embedding.astype(jnp.float32)

    ax2 = jnp.sum(audio.reshape(-1, _D) ** 2, axis=1)
    vx2 = jnp.sum(video.reshape(-1, _D) ** 2, axis=1)
    x2 = jnp.stack([ax2, vx2]).reshape(2, _B, _T)
    e2 = jnp.sum(emb ** 2, axis=1).reshape(1, _M)

    idx, ph, cnt, dw, bs = _stage_a(audio, video, emb, x2, e2)

    emb_pad = jnp.pad(emb, ((0, 0), (0, _DP - _D)))
    q = _sc_gather(emb_pad, idx.reshape(-1))[:, :_D]
    aq = q[:_N].reshape(_B, _T, _D)
    vq = q[_N:].reshape(_B, _T, _D)

    emb2, scal = _stage_c(cnt, dw, ph, bs,
                          ema_count.reshape(1, _M).astype(jnp.float32),
                          ema_weight.astype(jnp.float32),
                          audio, video, aq, vq)

    a_loss = scal[0, 0]
    v_loss = scal[0, 1]
    cmcm_loss = jnp.where(epoch < 10, 0.0, 0.5 * scal[0, 2])
    a_cons = scal[0, 3]
    v_cons = scal[0, 4]
    equal_num = scal[0, 5].astype(jnp.int32)

    return (aq, vq, a_loss, v_loss, cmcm_loss, a_cons, v_cons,
            equal_num, emb2)


# speedup vs baseline: 1.3705x; 1.3705x over previous
import functools

import jax
import jax.numpy as jnp
import numpy as np
from jax import lax
from jax.experimental import pallas as pl
from jax.experimental.pallas import tpu as pltpu
from jax.experimental.pallas import tpu_sc as plsc

_B, _T, _D, _M = 16, 128, 64, 8192
_N = _B * _T
_DECAY = 0.99
_EPS = 1e-05
_LOG_M = float(np.log(_M))


def _stage_a_body(a_ref, v_ref, e_ref, x2_ref, e2_ref,
                  idx_ref, ph_ref, cnt_ref, dw_ref, bs_ref):
    i = pl.program_id(0)
    mod = i // _B
    b = i % _B

    @pl.when(i == 0)
    def _init():
        cnt_ref[...] = jnp.zeros_like(cnt_ref)
        dw_ref[...] = jnp.zeros_like(dw_ref)

    a = a_ref[b]
    v = v_ref[b]
    x = jnp.where(mod == 0, a, v)
    sumflat = a + v
    e = e_ref[...]

    e2 = e2_ref[0]
    x2 = x2_ref[mod, b]
    xe = lax.dot_general(x, e, (((1,), (1,)), ((), ())),
                         preferred_element_type=jnp.float32)
    d = (e2[None, :] + x2[:, None]) - 2.0 * xe

    miota = lax.broadcasted_iota(jnp.int32, (_T, _M), 1)
    dmin = jnp.min(d, axis=1, keepdims=True)
    idx = jnp.min(jnp.where(d == dmin, miota, _M), axis=1).astype(jnp.int32)

    s = -jnp.sqrt(jnp.maximum(d, 0.0))
    smax = -jnp.sqrt(jnp.maximum(dmin, 0.0))
    u = jnp.exp(s - smax)
    z = jnp.sum(u, axis=1, keepdims=True)
    p = u / z
    ent = -jnp.sum(p * jnp.log(p + 1e-5), axis=1)
    adj = 1.0 - ent / _LOG_M
    ph_b = jnp.mean(p, axis=0)
    cons_b = jnp.sum(jnp.abs(p - ph_b[None, :]))

    one = (miota == idx[:, None]).astype(jnp.float32)
    cnt_vec = jnp.sum(one, axis=0)
    viota = lax.broadcasted_iota(jnp.int32, (1, _M), 1).reshape(_M)
    cmax = jnp.max(cnt_vec)
    mode = jnp.min(jnp.where(cnt_vec == cmax, viota, _M)).astype(jnp.float32)
    wcnt = jnp.sum(one * adj[:, None], axis=0)
    y = adj[:, None] * sumflat
    dwc = lax.dot_general(one, y, (((0,), (0,)), ((), ())),
                          preferred_element_type=jnp.float32)

    idx_ref[mod, b] = idx
    ph_ref[mod, b] = ph_b
    cnt_ref[mod] = cnt_ref[mod] + wcnt
    dw_ref[mod] = dw_ref[mod] + dwc
    li = lax.broadcasted_iota(jnp.int32, (1, _T), 1).reshape(_T)
    bs_ref[mod, b] = jnp.where(li == 0, cons_b,
                               jnp.where(li == 1, mode, 0.0))


def _stage_a(audio, video, embedding, x2, e2):
    full = lambda shape: pl.BlockSpec(shape, lambda i: (0,) * len(shape))
    return pl.pallas_call(
        _stage_a_body,
        grid=(2 * _B,),
        in_specs=[full((_B, _T, _D)), full((_B, _T, _D)), full((_M, _D)),
                  full((2, _B, _T)), full((1, _M))],
        out_specs=[full((2, _B, _T)), full((2, _B, _M)), full((2, _M)),
                   full((2, _M, _D)), full((2, _B, _T))],
        out_shape=[
            jax.ShapeDtypeStruct((2, _B, _T), jnp.int32),
            jax.ShapeDtypeStruct((2, _B, _M), jnp.float32),
            jax.ShapeDtypeStruct((2, _M), jnp.float32),
            jax.ShapeDtypeStruct((2, _M, _D), jnp.float32),
            jax.ShapeDtypeStruct((2, _B, _T), jnp.float32),
        ],
        compiler_params=pltpu.CompilerParams(
            dimension_semantics=("arbitrary",)),
    )(audio, video, embedding, x2, e2)


_NW = 32
_ROWS_PER_W = 2 * _N // _NW
_DP = 128


def _sc_gather(table_padded, idx_flat):
    mesh = plsc.VectorSubcoreMesh(core_axis_name="c", subcore_axis_name="s")

    @functools.partial(
        pl.kernel, mesh=mesh,
        out_type=jax.ShapeDtypeStruct((2 * _N, _DP), jnp.float32),
        scratch_types=[
            pltpu.VMEM((_ROWS_PER_W,), jnp.int32),
            pltpu.VMEM((_ROWS_PER_W, _DP), jnp.float32),
            pltpu.SemaphoreType.DMA,
        ],
    )
    def gather_k(table_hbm, idx_hbm, out_hbm, idx_v, rows_v, sem):
        wid = lax.axis_index("s") * 2 + lax.axis_index("c")
        base = wid * _ROWS_PER_W
        pltpu.sync_copy(idx_hbm.at[pl.ds(base, _ROWS_PER_W)], idx_v)
        pltpu.async_copy(table_hbm.at[idx_v], rows_v, sem).wait()
        pltpu.sync_copy(rows_v, out_hbm.at[pl.ds(base, _ROWS_PER_W)])

    return gather_k(table_padded, idx_flat)


def _stage_c_body(cnt_ref, dw_ref, ph_ref, bs_ref, ec_ref, ew_ref,
                  a_ref, v_ref, aq_ref, vq_ref,
                  emb_ref, sc_ref):
    ec = ec_ref[0]
    a_n = cnt_ref[0]
    v_n = cnt_ref[1]

    ec1 = _DECAY * ec + (1.0 - _DECAY) * v_n
    n1 = jnp.sum(ec1)
    ec1 = (ec1 + _EPS) / (n1 + _M * _EPS) * n1
    ew1 = _DECAY * ew_ref[...] + 0.5 * (1.0 - _DECAY) * dw_ref[1]
    ec2 = _DECAY * ec1 + (1.0 - _DECAY) * a_n
    n2 = jnp.sum(ec2)
    ec2 = (ec2 + _EPS) / (n2 + _M * _EPS) * n2
    ew2 = _DECAY * ew1 + 0.5 * (1.0 - _DECAY) * dw_ref[0]
    emb_ref[...] = ew2 / ec2[:, None]

    a_ph = ph_ref[0]
    v_ph = ph_ref[1]
    la = jnp.log(a_ph + 1e-10)
    lv = jnp.log(v_ph + 1e-10)
    scode = (lax.dot_general(a_ph, lv, (((1,), (1,)), ((), ())),
                             preferred_element_type=jnp.float32)
             + lax.dot_general(v_ph, la, (((1,), (1,)), ((), ())),
                               preferred_element_type=jnp.float32))
    mx = jnp.max(-scode)
    es = jnp.exp(scode + mx)
    rs = jnp.sum(es, axis=1)
    r = lax.broadcasted_iota(jnp.int32, (_B, _B), 0)
    c = lax.broadcasted_iota(jnp.int32, (_B, _B), 1)
    diag = jnp.sum(jnp.where(r == c, es, 0.0), axis=1)
    lcmcm = -jnp.sum(jnp.log(diag / (rs + _EPS))) / _B

    bs = bs_ref[...]
    mi = lax.broadcasted_iota(jnp.int32, (2, _B, _T), 0)
    li = lax.broadcasted_iota(jnp.int32, (2, _B, _T), 2)
    a_cons = jnp.sum(jnp.where((li == 0) & (mi == 0), bs, 0.0)) / _B
    v_cons = jnp.sum(jnp.where((li == 0) & (mi == 1), bs, 0.0)) / _B
    modes = jnp.sum(jnp.where(li == 1, bs, 0.0), axis=2)
    eqn = jnp.sum(jnp.where(modes[0] == modes[1], 1.0, 0.0))

    a_loss = 0.5 * jnp.mean((a_ref[...] - aq_ref[...]) ** 2)
    v_loss = 0.5 * jnp.mean((v_ref[...] - vq_ref[...]) ** 2)

    lo = lax.broadcasted_iota(jnp.int32, (1, 128), 1)
    sc_ref[...] = jnp.where(lo == 0, a_loss,
                  jnp.where(lo == 1, v_loss,
                  jnp.where(lo == 2, lcmcm,
                  jnp.where(lo == 3, a_cons,
                  jnp.where(lo == 4, v_cons,
                  jnp.where(lo == 5, eqn, 0.0))))))


def _stage_c(cnt, dw, ph, bs, ema_count, ema_weight, audio, video, aq, vq):
    full = lambda shape: pl.BlockSpec(shape, lambda: (0,) * len(shape))
    return pl.pallas_call(
        _stage_c_body,
        in_specs=[full((2, _M)), full((2, _M, _D)), full((2, _B, _M)),
                  full((2, _B, _T)), full((1, _M)), full((_M, _D)),
                  full((_B, _T, _D)), full((_B, _T, _D)),
                  full((_B, _T, _D)), full((_B, _T, _D))],
        out_specs=[full((_M, _D)), full((1, 128))],
        out_shape=[
            jax.ShapeDtypeStruct((_M, _D), jnp.float32),
            jax.ShapeDtypeStruct((1, 128), jnp.float32),
        ],
    )(cnt, dw, ph, bs, ema_count, ema_weight, audio, video, aq, vq)


def kernel(audio_semantic, video_semantic, epoch, embedding, ema_count,
           ema_weight, coefficients):
    audio = audio_semantic.astype(jnp.float32)
    video = video_semantic.astype(jnp.float32)
    emb = embedding.astype(jnp.float32)

    ax2 = jnp.sum(audio.reshape(-1, _D) ** 2, axis=1)
    vx2 = jnp.sum(video.reshape(-1, _D) ** 2, axis=1)
    x2 = jnp.stack([ax2, vx2]).reshape(2, _B, _T)
    e2 = jnp.sum(emb ** 2, axis=1).reshape(1, _M)

    idx, ph, cnt, dw, bs = _stage_a(audio, video, emb, x2, e2)

    emb_pad = jnp.pad(emb, ((0, 0), (0, _DP - _D)))
    q = _sc_gather(emb_pad, idx.reshape(-1))[:, :_D]
    aq = q[:_N].reshape(_B, _T, _D)
    vq = q[_N:].reshape(_B, _T, _D)

    emb2, scal = _stage_c(cnt, dw, ph, bs,
                          ema_count.reshape(1, _M).astype(jnp.float32),
                          ema_weight.astype(jnp.float32),
                          audio, video, aq, vq)

    a_loss = scal[0, 0]
    v_loss = scal[0, 1]
    cmcm_loss = jnp.where(epoch < 10, 0.0, 0.5 * scal[0, 2])
    a_cons = scal[0, 3]
    v_cons = scal[0, 4]
    equal_num = scal[0, 5].astype(jnp.int32)

    return (aq, vq, a_loss, v_loss, cmcm_loss, a_cons, v_cons,
            equal_num, emb2)


# speedup vs baseline: 1.4157x; 1.0330x over previous
import functools

import jax
import jax.numpy as jnp
import numpy as np
from jax import lax
from jax.experimental import pallas as pl
from jax.experimental.pallas import tpu as pltpu
from jax.experimental.pallas import tpu_sc as plsc

_B, _T, _D, _M = 16, 128, 64, 8192
_N = _B * _T
_DECAY = 0.99
_EPS = 1e-05
_LOG_M = float(np.log(_M))


def _stage_a_body(a_ref, v_ref, e_ref, x2_ref, e2_ref,
                  idx_ref, ph_ref, cnt_ref, dw_ref, bs_ref):
    i = pl.program_id(0)
    mod = i // _B
    b = i % _B

    @pl.when(i == 0)
    def _init():
        cnt_ref[...] = jnp.zeros_like(cnt_ref)
        dw_ref[...] = jnp.zeros_like(dw_ref)

    a = a_ref[b]
    v = v_ref[b]
    x = jnp.where(mod == 0, a, v)
    sumflat = a + v
    e = e_ref[...]

    e2 = e2_ref[0]
    x2 = x2_ref[mod, b]
    xe = lax.dot_general(x, e, (((1,), (1,)), ((), ())),
                         preferred_element_type=jnp.float32)
    d = (e2[None, :] + x2[:, None]) - 2.0 * xe

    miota = lax.broadcasted_iota(jnp.int32, (_T, _M), 1)
    dmin = jnp.min(d, axis=1, keepdims=True)
    idx = jnp.min(jnp.where(d == dmin, miota, _M), axis=1).astype(jnp.int32)

    s = -jnp.sqrt(jnp.maximum(d, 0.0))
    smax = jnp.max(s, axis=1, keepdims=True)
    u = jnp.exp(s - smax)
    z = jnp.sum(u, axis=1, keepdims=True)
    p = u / z
    ent = -jnp.sum(p * jnp.log(p + 1e-5), axis=1)
    adj = 1.0 - ent / _LOG_M
    ph_b = jnp.mean(p, axis=0)
    cons_b = jnp.sum(jnp.abs(p - ph_b[None, :]))

    one = (miota == idx[:, None]).astype(jnp.float32)
    cnt_vec = jnp.sum(one, axis=0)
    viota = lax.broadcasted_iota(jnp.int32, (1, _M), 1).reshape(_M)
    cmax = jnp.max(cnt_vec)
    mode = jnp.min(jnp.where(cnt_vec == cmax, viota, _M)).astype(jnp.float32)
    wcnt = jnp.sum(one * adj[:, None], axis=0)
    y = adj[:, None] * sumflat
    dwc = lax.dot_general(one, y, (((0,), (0,)), ((), ())),
                          preferred_element_type=jnp.float32)

    idx_ref[mod, b] = idx
    ph_ref[mod, b] = ph_b
    cnt_ref[mod] = cnt_ref[mod] + wcnt
    dw_ref[mod] = dw_ref[mod] + dwc
    li = lax.broadcasted_iota(jnp.int32, (1, _T), 1).reshape(_T)
    bs_ref[mod, b] = jnp.where(li == 0, cons_b,
                               jnp.where(li == 1, mode, 0.0))


def _stage_a(audio, video, embedding, x2, e2):
    full = lambda shape: pl.BlockSpec(shape, lambda i: (0,) * len(shape))
    return pl.pallas_call(
        _stage_a_body,
        grid=(2 * _B,),
        in_specs=[full((_B, _T, _D)), full((_B, _T, _D)), full((_M, _D)),
                  full((2, _B, _T)), full((1, _M))],
        out_specs=[full((2, _B, _T)), full((2, _B, _M)), full((2, _M)),
                   full((2, _M, _D)), full((2, _B, _T))],
        out_shape=[
            jax.ShapeDtypeStruct((2, _B, _T), jnp.int32),
            jax.ShapeDtypeStruct((2, _B, _M), jnp.float32),
            jax.ShapeDtypeStruct((2, _M), jnp.float32),
            jax.ShapeDtypeStruct((2, _M, _D), jnp.float32),
            jax.ShapeDtypeStruct((2, _B, _T), jnp.float32),
        ],
        compiler_params=pltpu.CompilerParams(
            dimension_semantics=("arbitrary",)),
    )(audio, video, embedding, x2, e2)


_NW = 32
_ROWS_PER_W = 2 * _N // _NW
_DP = 128


def _sc_gather(table_padded, idx_flat):
    mesh = plsc.VectorSubcoreMesh(core_axis_name="c", subcore_axis_name="s")

    @functools.partial(
        pl.kernel, mesh=mesh,
        out_type=jax.ShapeDtypeStruct((2 * _N, _DP), jnp.float32),
        scratch_types=[
            pltpu.VMEM((_ROWS_PER_W,), jnp.int32),
            pltpu.VMEM((_ROWS_PER_W, _DP), jnp.float32),
            pltpu.SemaphoreType.DMA,
        ],
    )
    def gather_k(table_hbm, idx_hbm, out_hbm, idx_v, rows_v, sem):
        wid = lax.axis_index("s") * 2 + lax.axis_index("c")
        base = wid * _ROWS_PER_W
        pltpu.sync_copy(idx_hbm.at[pl.ds(base, _ROWS_PER_W)], idx_v)
        pltpu.async_copy(table_hbm.at[idx_v], rows_v, sem).wait()
        pltpu.sync_copy(rows_v, out_hbm.at[pl.ds(base, _ROWS_PER_W)])

    return gather_k(table_padded, idx_flat)


def _stage_c_body(cnt_ref, dw_ref, ph_ref, bs_ref, ec_ref, ew_ref,
                  a_ref, v_ref, aq_ref, vq_ref,
                  emb_ref, sc_ref):
    ec = ec_ref[0]
    a_n = cnt_ref[0]
    v_n = cnt_ref[1]

    ec1 = _DECAY * ec + (1.0 - _DECAY) * v_n
    n1 = jnp.sum(ec1)
    ec1 = (ec1 + _EPS) / (n1 + _M * _EPS) * n1
    ew1 = _DECAY * ew_ref[...] + 0.5 * (1.0 - _DECAY) * dw_ref[1]
    ec2 = _DECAY * ec1 + (1.0 - _DECAY) * a_n
    n2 = jnp.sum(ec2)
    ec2 = (ec2 + _EPS) / (n2 + _M * _EPS) * n2
    ew2 = _DECAY * ew1 + 0.5 * (1.0 - _DECAY) * dw_ref[0]
    emb_ref[...] = ew2 / ec2[:, None]

    a_ph = ph_ref[0]
    v_ph = ph_ref[1]
    la = jnp.log(a_ph + 1e-10)
    lv = jnp.log(v_ph + 1e-10)
    scode = (lax.dot_general(a_ph, lv, (((1,), (1,)), ((), ())),
                             preferred_element_type=jnp.float32)
             + lax.dot_general(v_ph, la, (((1,), (1,)), ((), ())),
                               preferred_element_type=jnp.float32))
    mx = jnp.max(-scode)
    es = jnp.exp(scode + mx)
    rs = jnp.sum(es, axis=1)
    r = lax.broadcasted_iota(jnp.int32, (_B, _B), 0)
    c = lax.broadcasted_iota(jnp.int32, (_B, _B), 1)
    diag = jnp.sum(jnp.where(r == c, es, 0.0), axis=1)
    lcmcm = -jnp.sum(jnp.log(diag / (rs + _EPS))) / _B

    bs = bs_ref[...]
    mi = lax.broadcasted_iota(jnp.int32, (2, _B, _T), 0)
    li = lax.broadcasted_iota(jnp.int32, (2, _B, _T), 2)
    a_cons = jnp.sum(jnp.where((li == 0) & (mi == 0), bs, 0.0)) / _B
    v_cons = jnp.sum(jnp.where((li == 0) & (mi == 1), bs, 0.0)) / _B
    modes = jnp.sum(jnp.where(li == 1, bs, 0.0), axis=2)
    eqn = jnp.sum(jnp.where(modes[0] == modes[1], 1.0, 0.0))

    a_loss = 0.5 * jnp.mean((a_ref[...] - aq_ref[...]) ** 2)
    v_loss = 0.5 * jnp.mean((v_ref[...] - vq_ref[...]) ** 2)

    lo = lax.broadcasted_iota(jnp.int32, (1, 128), 1)
    sc_ref[...] = jnp.where(lo == 0, a_loss,
                  jnp.where(lo == 1, v_loss,
                  jnp.where(lo == 2, lcmcm,
                  jnp.where(lo == 3, a_cons,
                  jnp.where(lo == 4, v_cons,
                  jnp.where(lo == 5, eqn, 0.0))))))


def _stage_c(cnt, dw, ph, bs, ema_count, ema_weight, audio, video, aq, vq):
    full = lambda shape: pl.BlockSpec(shape, lambda: (0,) * len(shape))
    return pl.pallas_call(
        _stage_c_body,
        in_specs=[full((2, _M)), full((2, _M, _D)), full((2, _B, _M)),
                  full((2, _B, _T)), full((1, _M)), full((_M, _D)),
                  full((_B, _T, _D)), full((_B, _T, _D)),
                  full((_B, _T, _D)), full((_B, _T, _D))],
        out_specs=[full((_M, _D)), full((1, 128))],
        out_shape=[
            jax.ShapeDtypeStruct((_M, _D), jnp.float32),
            jax.ShapeDtypeStruct((1, 128), jnp.float32),
        ],
    )(cnt, dw, ph, bs, ema_count, ema_weight, audio, video, aq, vq)


def kernel(audio_semantic, video_semantic, epoch, embedding, ema_count,
           ema_weight, coefficients):
    audio = audio_semantic.astype(jnp.float32)
    video = video_semantic.astype(jnp.float32)
    emb = embedding.astype(jnp.float32)

    ax2 = jnp.sum(audio.reshape(-1, _D) ** 2, axis=1)
    vx2 = jnp.sum(video.reshape(-1, _D) ** 2, axis=1)
    x2 = jnp.stack([ax2, vx2]).reshape(2, _B, _T)
    e2 = jnp.sum(emb ** 2, axis=1).reshape(1, _M)

    idx, ph, cnt, dw, bs = _stage_a(audio, video, emb, x2, e2)

    emb_pad = jnp.pad(emb, ((0, 0), (0, _DP - _D)))
    q = _sc_gather(emb_pad, idx.reshape(-1))[:, :_D]
    aq = q[:_N].reshape(_B, _T, _D)
    vq = q[_N:].reshape(_B, _T, _D)

    emb2, scal = _stage_c(cnt, dw, ph, bs,
                          ema_count.reshape(1, _M).astype(jnp.float32),
                          ema_weight.astype(jnp.float32),
                          audio, video, aq, vq)

    a_loss = scal[0, 0]
    v_loss = scal[0, 1]
    cmcm_loss = jnp.where(epoch < 10, 0.0, 0.5 * scal[0, 2])
    a_cons = scal[0, 3]
    v_cons = scal[0, 4]
    equal_num = scal[0, 5].astype(jnp.int32)

    return (aq, vq, a_loss, v_loss, cmcm_loss, a_cons, v_cons,
            equal_num, emb2)
